# Initial kernel scaffold; baseline (speedup 1.0000x reference)
import jax
import jax.numpy as jnp
from jax.experimental import pallas as pl


def kernel(hidden_states, router_w, gate_w, up_w, down_w):
    raise NotImplementedError("write your pallas kernel here")


# speedup vs baseline: 1.2309x; 1.2309x over previous
import functools

import jax
import jax.numpy as jnp
from jax.experimental import pallas as pl
from jax.experimental.pallas import tpu as pltpu


def _moe_dense_body(x_ref, rw_ref, gate_ref, up_ref, down_ref,
                    out_ref, logits_ref, wmat_ref):
    e = pl.program_id(1)
    f = pl.program_id(2)

    @pl.when((e == 0) & (f == 0))
    def _prologue():
        x = x_ref[...]
        logits = jax.lax.dot_general(
            x, rw_ref[...], (((1,), (1,)), ((), ())),
            preferred_element_type=jnp.float32)
        logits_ref[...] = logits
        m = jnp.max(logits, axis=1, keepdims=True)
        ex = jnp.exp(logits - m)
        p = ex / jnp.sum(ex, axis=1, keepdims=True)
        ne = p.shape[1]
        col = jax.lax.broadcasted_iota(jnp.int32, p.shape, 1)
        m1 = jnp.max(p, axis=1, keepdims=True)
        i1 = jnp.min(jnp.where(p == m1, col, ne), axis=1, keepdims=True)
        oh1 = col == i1
        p2 = jnp.where(oh1, -jnp.inf, p)
        m2 = jnp.max(p2, axis=1, keepdims=True)
        i2 = jnp.min(jnp.where(p2 == m2, col, ne), axis=1, keepdims=True)
        oh2 = col == i2
        wmat_ref[...] = jnp.where(oh1, m1, 0.0) + jnp.where(oh2, m2, 0.0)
        out_ref[...] = jnp.zeros_like(out_ref)

    x = x_ref[...]
    g = jax.lax.dot_general(x, gate_ref[0], (((1,), (1,)), ((), ())),
                            preferred_element_type=jnp.float32)
    u = jax.lax.dot_general(x, up_ref[0], (((1,), (1,)), ((), ())),
                            preferred_element_type=jnp.float32)
    h = (g * jax.lax.logistic(g)) * u
    y = jax.lax.dot_general(h, down_ref[0], (((1,), (1,)), ((), ())),
                            preferred_element_type=jnp.float32)
    col = jax.lax.broadcasted_iota(jnp.int32, wmat_ref.shape, 1)
    w_e = jnp.sum(jnp.where(col == e, wmat_ref[...], 0.0), axis=1,
                  keepdims=True)
    out_ref[...] += y * w_e


def kernel(hidden_states, router_w, gate_w, up_w, down_w):
    b, s, d = hidden_states.shape
    n = b * s
    ne, ff = gate_w.shape[0], gate_w.shape[1]
    x = hidden_states.reshape(n, d)

    t0 = min(1024, n)
    ft = min(256, ff)
    nt, nf = n // t0, ff // ft

    out, logits = pl.pallas_call(
        _moe_dense_body,
        grid=(nt, ne, nf),
        in_specs=[
            pl.BlockSpec((t0, d), lambda t, e, f: (t, 0)),
            pl.BlockSpec((ne, d), lambda t, e, f: (0, 0)),
            pl.BlockSpec((1, ft, d), lambda t, e, f: (e, f, 0)),
            pl.BlockSpec((1, ft, d), lambda t, e, f: (e, f, 0)),
            pl.BlockSpec((1, d, ft), lambda t, e, f: (e, 0, f)),
        ],
        out_specs=[
            pl.BlockSpec((t0, d), lambda t, e, f: (t, 0)),
            pl.BlockSpec((t0, ne), lambda t, e, f: (t, 0)),
        ],
        out_shape=[
            jax.ShapeDtypeStruct((n, d), jnp.float32),
            jax.ShapeDtypeStruct((n, ne), jnp.float32),
        ],
        scratch_shapes=[pltpu.VMEM((t0, ne), jnp.float32)],
        compiler_params=pltpu.CompilerParams(
            dimension_semantics=("arbitrary", "arbitrary", "arbitrary")),
    )(x, router_w, gate_w, up_w, down_w)

    return out.reshape(b, s, d), logits
